# baseline (speedup 1.0000x reference)
import functools

import jax
import jax.numpy as jnp
import numpy as np
from jax import lax
from jax.experimental import pallas as pl
from jax.experimental.pallas import tpu as pltpu
from jax.experimental.pallas import tpu_sc as plsc

HEAD_DIM = 128
BASE = 10000
FREQ_ALLOCATION = 20
HALF = HEAD_DIM // 2
SPLIT = HALF - FREQ_ALLOCATION

TBL_W = 48
SEQ = 8192
NPAIRS = 4 * SEQ
NWORK = 32
PW = NPAIRS // NWORK
CHUNK = 128
NCHUNK = PW // CHUNK
P_BLK = 2048


def _freq_kind_tables():
    inv_freq = 1.0 / (BASE ** (np.arange(0, HEAD_DIM, 2, dtype=np.float32) / HEAD_DIM))
    freqs = np.zeros((3, TBL_W), np.float32)
    kinds = np.full((3, TBL_W), 2, np.int32)
    sec_js = [
        np.arange(SPLIT, HALF),
        np.arange(0, SPLIT, 2),
        np.arange(1, SPLIT, 2),
    ]
    for s, js in enumerate(sec_js):
        n = len(js)
        freqs[s, :n] = inv_freq[js]
        freqs[s, n:2 * n] = inv_freq[js]
        kinds[s, :n] = 0
        kinds[s, n:2 * n] = 1
    return jnp.asarray(freqs)[:, None, :], jnp.asarray(kinds)[:, None, :]


def _off_tables():
    sec = np.zeros(256, np.int32)
    col = np.zeros(256, np.int32)
    for cp in range(256):
        trig, c = divmod(cp, HEAD_DIM)
        j = c // 2
        if j < SPLIT:
            if j % 2 == 0:
                sec[cp], col[cp] = 1, j // 2 + 22 * trig
            else:
                sec[cp], col[cp] = 2, (j - 1) // 2 + 22 * trig
        else:
            sec[cp], col[cp] = 0, (j - SPLIT) + 20 * trig
    return jnp.asarray(np.stack([sec, col]))


def _table_body(freq_ref, kind_ref, out_ref):
    i = pl.program_id(1)
    p = (lax.broadcasted_iota(jnp.int32, (P_BLK, TBL_W), 0) + i * P_BLK).astype(jnp.float32)
    ang = p * freq_ref[0]
    kind = kind_ref[0]
    out_ref[0] = jnp.where(kind == 0, jnp.sin(ang),
                           jnp.where(kind == 1, jnp.cos(ang), 0.0))


def _build_table(freqs, kinds):
    return pl.pallas_call(
        _table_body,
        grid=(3, SEQ // P_BLK),
        in_specs=[
            pl.BlockSpec((1, 1, TBL_W), lambda s, i: (s, 0, 0)),
            pl.BlockSpec((1, 1, TBL_W), lambda s, i: (s, 0, 0)),
        ],
        out_specs=pl.BlockSpec((1, P_BLK, TBL_W), lambda s, i: (s, i, 0)),
        out_shape=jax.ShapeDtypeStruct((3, SEQ, TBL_W), jnp.float32),
    )(freqs, kinds)


def _sc_gather_kernel(table_hbm, pid_hbm, off_hbm, out_hbm,
                      pid_v, idx_t, idx_h, idx_w, gbuf, obuf, offv, sem):
    wid = lax.axis_index("s") * 2 + lax.axis_index("c")
    pltpu.sync_copy(off_hbm, offv)
    i3 = lax.broadcasted_iota(jnp.int32, (16,), 0) * 3

    def chunk_body(chk, carry):
        gbase = wid * PW + chk * CHUNK
        pltpu.sync_copy(pid_hbm.at[pl.ds(gbase * 3, 3 * CHUNK)], pid_v)
        for v in range(CHUNK // 16):
            g = i3 + v * 48
            tv = plsc.load_gather(pid_v, [g])
            hv = plsc.load_gather(pid_v, [g + 1])
            wv = plsc.load_gather(pid_v, [g + 2])
            idx_t[pl.ds(v * 16, 16)] = tv
            idx_h[pl.ds(v * 16, 16)] = hv + SEQ
            idx_w[pl.ds(v * 16, 16)] = wv + 2 * SEQ
        ct = pltpu.async_copy(table_hbm.at[idx_t], gbuf.at[0], sem)
        ch = pltpu.async_copy(table_hbm.at[idx_h], gbuf.at[1], sem)
        cw = pltpu.async_copy(table_hbm.at[idx_w], gbuf.at[2], sem)
        ct.wait()
        ch.wait()
        cw.wait()

        def pair_body(k, carry2):
            rowv = jnp.full((16,), 0, jnp.int32) + k
            for v in range(16):
                secv = offv[0, pl.ds(v * 16, 16)]
                colv = offv[1, pl.ds(v * 16, 16)]
                vals = plsc.load_gather(gbuf, [secv, rowv, colv])
                obuf[v // 8, pl.ds(k * HEAD_DIM + (v % 8) * 16, 16)] = vals
            return carry2

        lax.fori_loop(0, CHUNK, pair_body, 0)
        pltpu.sync_copy(obuf.at[0], out_hbm.at[0, pl.ds(gbase * HEAD_DIM, CHUNK * HEAD_DIM)])
        pltpu.sync_copy(obuf.at[1], out_hbm.at[1, pl.ds(gbase * HEAD_DIM, CHUNK * HEAD_DIM)])
        return carry

    lax.fori_loop(0, NCHUNK, chunk_body, 0)


def kernel(position_ids):
    B, S, _ = position_ids.shape
    freqs, kinds = _freq_kind_tables()
    table = _build_table(freqs, kinds).reshape(3 * SEQ, TBL_W)
    offs = _off_tables()
    pid_flat = position_ids.reshape(-1)

    sc = functools.partial(
        pl.kernel,
        mesh=plsc.VectorSubcoreMesh(core_axis_name="c", subcore_axis_name="s"),
        out_type=jax.ShapeDtypeStruct((2, NPAIRS * HEAD_DIM), jnp.float32),
        scratch_types=[
            pltpu.VMEM((3 * CHUNK,), jnp.int32),
            pltpu.VMEM((CHUNK,), jnp.int32),
            pltpu.VMEM((CHUNK,), jnp.int32),
            pltpu.VMEM((CHUNK,), jnp.int32),
            pltpu.VMEM((3, CHUNK, TBL_W), jnp.float32),
            pltpu.VMEM((2, CHUNK * HEAD_DIM), jnp.float32),
            pltpu.VMEM((2, 256), jnp.int32),
            pltpu.SemaphoreType.DMA,
        ],
        compiler_params=pltpu.CompilerParams(
            needs_layout_passes=False, use_tc_tiling_on_sc=False),
    )(_sc_gather_kernel)
    out = sc(table, pid_flat, offs)
    return out.reshape(2 * B, S, 1, HEAD_DIM)


# speedup vs baseline: 1.6201x; 1.6201x over previous
import functools

import jax
import jax.numpy as jnp
import numpy as np
from jax import lax
from jax.experimental import pallas as pl
from jax.experimental.pallas import tpu as pltpu
from jax.experimental.pallas import tpu_sc as plsc

HEAD_DIM = 128
BASE = 10000
FREQ_ALLOCATION = 20
HALF = HEAD_DIM // 2
SPLIT = HALF - FREQ_ALLOCATION

TBL_W = 48
SEQ = 8192
NPAIRS = 4 * SEQ
NWORK = 32
PW = NPAIRS // NWORK
CHUNK = 128
NCHUNK = PW // CHUNK
P_BLK = 2048


def _freq_kind_tables():
    inv_freq = 1.0 / (BASE ** (np.arange(0, HEAD_DIM, 2, dtype=np.float32) / HEAD_DIM))
    freqs = np.zeros((3, TBL_W), np.float32)
    kinds = np.full((3, TBL_W), 2, np.int32)
    sec_js = [
        np.arange(SPLIT, HALF),
        np.arange(0, SPLIT, 2),
        np.arange(1, SPLIT, 2),
    ]
    for s, js in enumerate(sec_js):
        n = len(js)
        freqs[s, :n] = inv_freq[js]
        freqs[s, n:2 * n] = inv_freq[js]
        kinds[s, :n] = 0
        kinds[s, n:2 * n] = 1
    return jnp.asarray(freqs)[:, None, :], jnp.asarray(kinds)[:, None, :]


def _off_tables():
    off = np.zeros(256, np.int32)
    for cp in range(256):
        trig, c = divmod(cp, HEAD_DIM)
        j = c // 2
        if j < SPLIT:
            if j % 2 == 0:
                sec, col = 1, j // 2 + 22 * trig
            else:
                sec, col = 2, (j - 1) // 2 + 22 * trig
        else:
            sec, col = 0, (j - SPLIT) + 20 * trig
        off[cp] = sec * CHUNK * TBL_W + col
    return jnp.asarray(off)


def _table_body(freq_ref, kind_ref, out_ref):
    i = pl.program_id(1)
    p = (lax.broadcasted_iota(jnp.int32, (P_BLK, TBL_W), 0) + i * P_BLK).astype(jnp.float32)
    ang = p * freq_ref[0]
    kind = kind_ref[0]
    out_ref[0] = jnp.where(kind == 0, jnp.sin(ang),
                           jnp.where(kind == 1, jnp.cos(ang), 0.0))


def _build_table(freqs, kinds):
    return pl.pallas_call(
        _table_body,
        grid=(3, SEQ // P_BLK),
        in_specs=[
            pl.BlockSpec((1, 1, TBL_W), lambda s, i: (s, 0, 0)),
            pl.BlockSpec((1, 1, TBL_W), lambda s, i: (s, 0, 0)),
        ],
        out_specs=pl.BlockSpec((1, P_BLK, TBL_W), lambda s, i: (s, i, 0)),
        out_shape=jax.ShapeDtypeStruct((3, SEQ, TBL_W), jnp.float32),
    )(freqs, kinds)


def _sc_gather_kernel(table_hbm, pid_hbm, off_hbm, out_hbm,
                      pid_v, idx_t, idx_h, idx_w, gbuf, obuf, offv, sem):
    wid = lax.axis_index("s") * 2 + lax.axis_index("c")
    pltpu.sync_copy(off_hbm, offv)
    i16 = lax.broadcasted_iota(jnp.int32, (16,), 0)
    i3 = i16 * 3
    zero = i16 * 0

    def chunk_body(chk, carry):
        gbase = wid * PW + chk * CHUNK
        pltpu.sync_copy(pid_hbm.at[pl.ds(gbase * 3, 3 * CHUNK)], pid_v)
        for v in range(CHUNK // 16):
            g = i3 + v * 48
            tv = plsc.load_gather(pid_v, [g])
            hv = plsc.load_gather(pid_v, [g + 1])
            wv = plsc.load_gather(pid_v, [g + 2])
            idx_t[pl.ds(v * 16, 16)] = tv
            idx_h[pl.ds(v * 16, 16)] = hv + SEQ
            idx_w[pl.ds(v * 16, 16)] = wv + 2 * SEQ
        ct = pltpu.async_copy(table_hbm.at[idx_t], gbuf.at[pl.ds(0, CHUNK)], sem)
        ch = pltpu.async_copy(table_hbm.at[idx_h], gbuf.at[pl.ds(CHUNK, CHUNK)], sem)
        cw = pltpu.async_copy(table_hbm.at[idx_w], gbuf.at[pl.ds(2 * CHUNK, CHUNK)], sem)
        ct.wait()
        ch.wait()
        cw.wait()

        def pair_body(k, offs):
            for v in range(16):
                vals = plsc.load_gather(gbuf, [zero, offs[v]])
                obuf[v // 8, pl.ds(k * HEAD_DIM + (v % 8) * 16, 16)] = vals
            return tuple(o + TBL_W for o in offs)

        offs0 = tuple(offv[pl.ds(v * 16, 16)] for v in range(16))
        lax.fori_loop(0, CHUNK, pair_body, offs0, unroll=4)
        pltpu.sync_copy(obuf.at[0], out_hbm.at[0, pl.ds(gbase * HEAD_DIM, CHUNK * HEAD_DIM)])
        pltpu.sync_copy(obuf.at[1], out_hbm.at[1, pl.ds(gbase * HEAD_DIM, CHUNK * HEAD_DIM)])
        return carry

    lax.fori_loop(0, NCHUNK, chunk_body, 0)


def kernel(position_ids):
    B, S, _ = position_ids.shape
    freqs, kinds = _freq_kind_tables()
    table = _build_table(freqs, kinds).reshape(3 * SEQ, TBL_W)
    offs = _off_tables()
    pid_flat = position_ids.reshape(-1)

    sc = functools.partial(
        pl.kernel,
        mesh=plsc.VectorSubcoreMesh(core_axis_name="c", subcore_axis_name="s"),
        out_type=jax.ShapeDtypeStruct((2, NPAIRS * HEAD_DIM), jnp.float32),
        scratch_types=[
            pltpu.VMEM((3 * CHUNK,), jnp.int32),
            pltpu.VMEM((CHUNK,), jnp.int32),
            pltpu.VMEM((CHUNK,), jnp.int32),
            pltpu.VMEM((CHUNK,), jnp.int32),
            pltpu.VMEM((3 * CHUNK, TBL_W), jnp.float32),
            pltpu.VMEM((2, CHUNK * HEAD_DIM), jnp.float32),
            pltpu.VMEM((256,), jnp.int32),
            pltpu.SemaphoreType.DMA,
        ],
        compiler_params=pltpu.CompilerParams(
            needs_layout_passes=False, use_tc_tiling_on_sc=False),
    )(_sc_gather_kernel)
    out = sc(table, pid_flat, offs)
    return out.reshape(2 * B, S, 1, HEAD_DIM)


# speedup vs baseline: 2.3179x; 1.4307x over previous
import functools

import jax
import jax.numpy as jnp
import numpy as np
from jax import lax
from jax.experimental import pallas as pl
from jax.experimental.pallas import tpu as pltpu
from jax.experimental.pallas import tpu_sc as plsc

HEAD_DIM = 128
BASE = 10000
FREQ_ALLOCATION = 20
HALF = HEAD_DIM // 2
SPLIT = HALF - FREQ_ALLOCATION

TBL_W = 48
SEQ = 8192
NPAIRS = 4 * SEQ
NWORK = 32
PW = NPAIRS // NWORK
CHUNK = 128
NCHUNK = PW // CHUNK
P_BLK = 2048


def _freq_kind_tables():
    inv_freq = 1.0 / (BASE ** (np.arange(0, HEAD_DIM, 2, dtype=np.float32) / HEAD_DIM))
    freqs = np.zeros((3, TBL_W), np.float32)
    kinds = np.full((3, TBL_W), 2, np.int32)
    sec_js = [
        np.arange(SPLIT, HALF),
        np.arange(0, SPLIT, 2),
        np.arange(1, SPLIT, 2),
    ]
    for s, js in enumerate(sec_js):
        n = len(js)
        freqs[s, :n] = inv_freq[js]
        freqs[s, n:2 * n] = inv_freq[js]
        kinds[s, :n] = 0
        kinds[s, n:2 * n] = 1
    return jnp.asarray(freqs)[:, None, :], jnp.asarray(kinds)[:, None, :]


def _off_tables():
    off = np.zeros(256, np.int32)
    for cp in range(256):
        trig, c = divmod(cp, HEAD_DIM)
        j = c // 2
        if j < SPLIT:
            if j % 2 == 0:
                sec, col = 1, j // 2 + 22 * trig
            else:
                sec, col = 2, (j - 1) // 2 + 22 * trig
        else:
            sec, col = 0, (j - SPLIT) + 20 * trig
        off[cp] = sec * CHUNK * TBL_W + col
    return jnp.asarray(off)


def _table_body(freq_ref, kind_ref, out_ref):
    i = pl.program_id(1)
    p = (lax.broadcasted_iota(jnp.int32, (P_BLK, TBL_W), 0) + i * P_BLK).astype(jnp.float32)
    ang = p * freq_ref[0]
    kind = kind_ref[0]
    out_ref[0] = jnp.where(kind == 0, jnp.sin(ang),
                           jnp.where(kind == 1, jnp.cos(ang), 0.0))


def _build_table(freqs, kinds):
    return pl.pallas_call(
        _table_body,
        grid=(3, SEQ // P_BLK),
        in_specs=[
            pl.BlockSpec((1, 1, TBL_W), lambda s, i: (s, 0, 0)),
            pl.BlockSpec((1, 1, TBL_W), lambda s, i: (s, 0, 0)),
        ],
        out_specs=pl.BlockSpec((1, P_BLK, TBL_W), lambda s, i: (s, i, 0)),
        out_shape=jax.ShapeDtypeStruct((3, SEQ, TBL_W), jnp.float32),
    )(freqs, kinds)


def _sc_gather_kernel(table_hbm, pid_hbm, off_hbm, out_hbm,
                      pid_v, idx_t, idx_h, idx_w, gbuf, obuf, offv, sem):
    wid = lax.axis_index("s") * 2 + lax.axis_index("c")
    pltpu.sync_copy(off_hbm, offv)
    i16 = lax.broadcasted_iota(jnp.int32, (16,), 0)
    i3 = i16 * 3
    zero = i16 * 0

    def chunk_body(chk, carry):
        gbase = wid * PW + chk * CHUNK
        pltpu.sync_copy(pid_hbm.at[pl.ds(gbase * 3, 3 * CHUNK)], pid_v)
        for v in range(CHUNK // 16):
            g = i3 + v * 48
            tv = plsc.load_gather(pid_v, [g])
            hv = plsc.load_gather(pid_v, [g + 1])
            wv = plsc.load_gather(pid_v, [g + 2])
            idx_t[pl.ds(v * 16, 16)] = tv
            idx_h[pl.ds(v * 16, 16)] = hv + SEQ
            idx_w[pl.ds(v * 16, 16)] = wv + 2 * SEQ
        ct = pltpu.async_copy(table_hbm.at[idx_t], gbuf.at[pl.ds(0, CHUNK)], sem)
        ch = pltpu.async_copy(table_hbm.at[idx_h], gbuf.at[pl.ds(CHUNK, CHUNK)], sem)
        cw = pltpu.async_copy(table_hbm.at[idx_w], gbuf.at[pl.ds(2 * CHUNK, CHUNK)], sem)
        ct.wait()
        ch.wait()
        cw.wait()

        offs0 = tuple(offv[pl.ds(v * 16, 16)] for v in range(16))

        @plsc.parallel_loop(0, CHUNK, carry=offs0, unroll=4)
        def pair_body(k, offs):
            for v in range(16):
                vals = plsc.load_gather(gbuf, [zero, offs[v]])
                obuf[v // 8, pl.ds(k * HEAD_DIM + (v % 8) * 16, 16)] = vals
            return tuple(o + TBL_W for o in offs)
        pltpu.sync_copy(obuf.at[0], out_hbm.at[0, pl.ds(gbase * HEAD_DIM, CHUNK * HEAD_DIM)])
        pltpu.sync_copy(obuf.at[1], out_hbm.at[1, pl.ds(gbase * HEAD_DIM, CHUNK * HEAD_DIM)])
        return carry

    lax.fori_loop(0, NCHUNK, chunk_body, 0)


def kernel(position_ids):
    B, S, _ = position_ids.shape
    freqs, kinds = _freq_kind_tables()
    table = _build_table(freqs, kinds).reshape(3 * SEQ, TBL_W)
    offs = _off_tables()
    pid_flat = position_ids.reshape(-1)

    sc = functools.partial(
        pl.kernel,
        mesh=plsc.VectorSubcoreMesh(core_axis_name="c", subcore_axis_name="s"),
        out_type=jax.ShapeDtypeStruct((2, NPAIRS * HEAD_DIM), jnp.float32),
        scratch_types=[
            pltpu.VMEM((3 * CHUNK,), jnp.int32),
            pltpu.VMEM((CHUNK,), jnp.int32),
            pltpu.VMEM((CHUNK,), jnp.int32),
            pltpu.VMEM((CHUNK,), jnp.int32),
            pltpu.VMEM((3 * CHUNK, TBL_W), jnp.float32),
            pltpu.VMEM((2, CHUNK * HEAD_DIM), jnp.float32),
            pltpu.VMEM((256,), jnp.int32),
            pltpu.SemaphoreType.DMA,
        ],
        compiler_params=pltpu.CompilerParams(
            needs_layout_passes=False, use_tc_tiling_on_sc=False),
    )(_sc_gather_kernel)
    out = sc(table, pid_flat, offs)
    return out.reshape(2 * B, S, 1, HEAD_DIM)


# speedup vs baseline: 3.3130x; 1.4293x over previous
import functools

import jax
import jax.numpy as jnp
import numpy as np
from jax import lax
from jax.experimental import pallas as pl
from jax.experimental.pallas import tpu as pltpu
from jax.experimental.pallas import tpu_sc as plsc

HEAD_DIM = 128
BASE = 10000
FREQ_ALLOCATION = 20
HALF = HEAD_DIM // 2
SPLIT = HALF - FREQ_ALLOCATION

SEQ = 8192
NPAIRS = 4 * SEQ
NWORK = 32
PW = NPAIRS // NWORK
CHUNK = 64
NCHUNK = PW // CHUNK
T_BLK = 2048


def _freq_phase_tables():
    inv_freq = 1.0 / (BASE ** (np.arange(0, HEAD_DIM, 2, dtype=np.float32) / HEAD_DIM))
    freq = np.zeros(HEAD_DIM, np.float32)
    phase = np.zeros(HEAD_DIM, np.float32)
    hp = np.float32(np.pi / 2)
    segs = [
        (0, np.arange(SPLIT, HALF), 0.0),
        (20, np.arange(SPLIT, HALF), hp),
        (40, np.arange(0, SPLIT, 2), 0.0),
        (62, np.arange(0, SPLIT, 2), hp),
        (84, np.arange(1, SPLIT, 2), 0.0),
        (106, np.arange(1, SPLIT, 2), hp),
    ]
    for base, js, ph in segs:
        freq[base:base + len(js)] = inv_freq[js]
        phase[base:base + len(js)] = ph
    return jnp.asarray(freq)[None], jnp.asarray(phase)[None]


def _off_table():
    off = np.zeros(256, np.int32)
    for cp in range(256):
        trig, c = divmod(cp, HEAD_DIM)
        j = c // 2
        if j >= SPLIT:
            sec, col = 0, (j - SPLIT) + 20 * trig
        elif j % 2 == 0:
            sec, col = 1, 40 + j // 2 + 22 * trig
        else:
            sec, col = 2, 84 + (j - 1) // 2 + 22 * trig
        off[cp] = sec * CHUNK * HEAD_DIM + col
    return jnp.asarray(off)


def _table_body(freq_ref, phase_ref, out_ref):
    i = pl.program_id(0)
    p = (lax.broadcasted_iota(jnp.int32, (T_BLK, HEAD_DIM), 0) + i * T_BLK)
    out_ref[...] = jnp.sin(p.astype(jnp.float32) * freq_ref[...] + phase_ref[...])


def _build_table(freqs, phases):
    return pl.pallas_call(
        _table_body,
        grid=(SEQ // T_BLK,),
        in_specs=[
            pl.BlockSpec((1, HEAD_DIM), lambda i: (0, 0)),
            pl.BlockSpec((1, HEAD_DIM), lambda i: (0, 0)),
        ],
        out_specs=pl.BlockSpec((T_BLK, HEAD_DIM), lambda i: (i, 0)),
        out_shape=jax.ShapeDtypeStruct((SEQ, HEAD_DIM), jnp.float32),
    )(freqs, phases)


def _sc_gather_kernel(table_hbm, pid_hbm, off_hbm, out_hbm,
                      pid_v, idx_t, idx_h, idx_w, gbuf, obuf, offv,
                      sem_a, sem_b):
    wid = lax.axis_index("s") * 2 + lax.axis_index("c")
    b = wid // 8
    srow = (wid % 8) * PW
    pltpu.sync_copy(off_hbm, offv)
    i16 = lax.broadcasted_iota(jnp.int32, (16,), 0)
    i3 = i16 * 3
    zero = i16 * 0
    sems = (sem_a, sem_b)

    def build_and_fire(chk, par, sem):
        gbase = wid * PW + chk * CHUNK
        pltpu.sync_copy(pid_hbm.at[pl.ds(gbase * 3, 3 * CHUNK)], pid_v)
        for v in range(CHUNK // 16):
            g = i3 + v * 48
            idx_t[pl.ds(v * 16, 16)] = plsc.load_gather(pid_v, [g])
            idx_h[pl.ds(v * 16, 16)] = plsc.load_gather(pid_v, [g + 1])
            idx_w[pl.ds(v * 16, 16)] = plsc.load_gather(pid_v, [g + 2])
        base = par * 3 * CHUNK
        pltpu.async_copy(table_hbm.at[idx_t], gbuf.at[pl.ds(base, CHUNK)], sem)
        pltpu.async_copy(table_hbm.at[idx_h], gbuf.at[pl.ds(base + CHUNK, CHUNK)], sem)
        pltpu.async_copy(table_hbm.at[idx_w], gbuf.at[pl.ds(base + 2 * CHUNK, CHUNK)], sem)

    def wait_gathers(par, sem):
        base = par * 3 * CHUNK
        for s in range(3):
            pltpu.make_async_copy(
                table_hbm.at[idx_t],
                gbuf.at[pl.ds(base + s * CHUNK, CHUNK)], sem).wait()

    build_and_fire(0, 0, sem_a)

    def loop_body(i2, carry):
        for sub in range(2):
            chk = i2 * 2 + sub
            wait_gathers(sub, sems[sub])

            @pl.when(chk + 1 < NCHUNK)
            def _():
                build_and_fire(chk + 1, 1 - sub, sems[1 - sub])

            offs0 = tuple(offv[pl.ds(v * 16, 16)] + sub * 3 * CHUNK * HEAD_DIM
                          for v in range(16))

            @plsc.parallel_loop(0, CHUNK, carry=offs0, unroll=4)
            def pair_body(k, offs):
                for v in range(16):
                    vals = plsc.load_gather(gbuf, [zero, offs[v]])
                    obuf[v // 8, k, pl.ds((v % 8) * 16, 16)] = vals
                return tuple(o + HEAD_DIM for o in offs)

            s0 = srow + chk * CHUNK
            pltpu.sync_copy(obuf.at[0], out_hbm.at[b, pl.ds(s0, CHUNK), 0])
            pltpu.sync_copy(obuf.at[1], out_hbm.at[b + 4, pl.ds(s0, CHUNK), 0])
        return carry

    lax.fori_loop(0, NCHUNK // 2, loop_body, 0)


def kernel(position_ids):
    B, S, _ = position_ids.shape
    freqs, phases = _freq_phase_tables()
    table = _build_table(freqs, phases)
    offs = _off_table()
    pid_flat = position_ids.reshape(-1)

    sc = functools.partial(
        pl.kernel,
        mesh=plsc.VectorSubcoreMesh(core_axis_name="c", subcore_axis_name="s"),
        out_type=jax.ShapeDtypeStruct((2 * B, S, 1, HEAD_DIM), jnp.float32),
        scratch_types=[
            pltpu.VMEM((3 * CHUNK,), jnp.int32),
            pltpu.VMEM((CHUNK,), jnp.int32),
            pltpu.VMEM((CHUNK,), jnp.int32),
            pltpu.VMEM((CHUNK,), jnp.int32),
            pltpu.VMEM((2 * 3 * CHUNK, HEAD_DIM), jnp.float32),
            pltpu.VMEM((2, CHUNK, HEAD_DIM), jnp.float32),
            pltpu.VMEM((256,), jnp.int32),
            pltpu.SemaphoreType.DMA,
            pltpu.SemaphoreType.DMA,
        ],
        compiler_params=pltpu.CompilerParams(
            needs_layout_passes=False, use_tc_tiling_on_sc=False),
    )(_sc_gather_kernel)
    return sc(table, pid_flat, offs)
